# baseline (speedup 1.0000x reference)
import numpy as np
import jax
import jax.numpy as jnp
from jax.experimental import pallas as pl

_POOLING_COUNTS = [15, 8, 8]
_N = 64


def _mask2d_np():
    mask = np.zeros((_N, _N), dtype=bool)
    mask[np.arange(_N), np.arange(_N)] = True
    stride, offset = 1, 0
    for c in _POOLING_COUNTS:
        for _ in range(c):
            offset += stride
            i = np.arange(0, _N - offset)
            mask[i, i + offset] = True
        stride *= 2
    return mask


def _body(start_ref, end_ref, m1_ref, m2_ref, out_ref):
    D = start_ref.shape[0]
    s = start_ref[...]
    e = end_ref[...]
    out_ref[:D] = jnp.dot(s, m1_ref[...], preferred_element_type=jnp.float32)
    out_ref[D:] = jnp.dot(e, m2_ref[...], preferred_element_type=jnp.float32)


def kernel(start, end):
    B, D, N = start.shape
    mask_np = _mask2d_np()
    ii, jj = np.nonzero(mask_np)
    m1_np = np.zeros((N, N * N), dtype=np.float32)
    m1_np[ii, ii * N + jj] = 1.0
    m2_np = np.zeros((N, N * N), dtype=np.float32)
    m2_np[jj, ii * N + jj] = 1.0
    m1 = jnp.asarray(m1_np)
    m2 = jnp.asarray(m2_np)
    flat = pl.pallas_call(
        _body,
        grid=(B,),
        in_specs=[
            pl.BlockSpec((None, D, N), lambda b: (b, 0, 0)),
            pl.BlockSpec((None, D, N), lambda b: (b, 0, 0)),
            pl.BlockSpec((N, N * N), lambda b: (0, 0)),
            pl.BlockSpec((N, N * N), lambda b: (0, 0)),
        ],
        out_specs=pl.BlockSpec((None, 2 * D, N * N), lambda b: (b, 0, 0)),
        out_shape=jax.ShapeDtypeStruct((B, 2 * D, N * N), start.dtype),
    )(start, end, m1, m2)
    return flat.reshape(B, 2 * D, N, N), jnp.asarray(mask_np)
